# Initial kernel scaffold; baseline (speedup 1.0000x reference)
import jax
import jax.numpy as jnp
from jax.experimental import pallas as pl


def kernel(x, conv_w, gamma, beta):
    raise NotImplementedError("write your pallas kernel here")


# speedup vs baseline: 3.3841x; 3.3841x over previous
import functools

import jax
import jax.numpy as jnp
from jax.experimental import pallas as pl
from jax.experimental.pallas import tpu as pltpu

_EPS = 1e-5


def _shift_lanes(x, k):
    return pltpu.roll(x, k % x.shape[1], 1)


def _conv_image(x, w_ref, m_ref, hw):
    xm = _shift_lanes(x, 1) * m_ref[0]
    xp = _shift_lanes(x, -1) * m_ref[1]
    xcat = jnp.concatenate([xm, x, xp], axis=0)

    a0 = jnp.dot(w_ref[1], xcat, preferred_element_type=jnp.float32)
    am = jnp.dot(w_ref[0], xcat, preferred_element_type=jnp.float32)
    ap = jnp.dot(w_ref[2], xcat, preferred_element_type=jnp.float32)

    h, w = hw
    p = h * w
    cout = a0.shape[0]
    z = jnp.zeros((cout, w), jnp.float32)
    return (a0
            + jnp.concatenate([z, am[:, :p - w]], axis=1)
            + jnp.concatenate([ap[:, w:], z], axis=1))


def _stats_kernel(x_ref, w_ref, m_ref, s1_ref, s2_ref, *, hw):
    y = _conv_image(x_ref[0], w_ref, m_ref, hw)
    cout = y.shape[0]
    s1 = jnp.sum(y, axis=1, keepdims=True)
    s2 = jnp.sum(y * y, axis=1, keepdims=True)
    s1_ref[0] = jnp.broadcast_to(s1, (cout, 128))
    s2_ref[0] = jnp.broadcast_to(s2, (cout, 128))


def _apply_kernel(x_ref, w_ref, m_ref, shift_ref, o_ref, *, hw):
    y = _conv_image(x_ref[0], w_ref, m_ref, hw)
    o_ref[0] = y + jnp.broadcast_to(shift_ref[:, 0:1], y.shape)


def kernel(x, conv_w, gamma, beta):
    n, cin, h, w = x.shape
    cout = conv_w.shape[0]
    p = h * w
    m_total = n * p

    x3 = x.reshape(n, cin, p)
    w3 = jnp.transpose(conv_w, (2, 0, 3, 1)).reshape(3, cout, 3 * cin)

    wcol = jnp.arange(p, dtype=jnp.int32) % w
    masks = jnp.stack([(wcol != 0), (wcol != w - 1)]).astype(jnp.float32)
    masks = jnp.broadcast_to(masks[:, None, :], (2, cin, p))

    hw = (h, w)
    stats1, stats2 = pl.pallas_call(
        functools.partial(_stats_kernel, hw=hw),
        out_shape=(
            jax.ShapeDtypeStruct((n, cout, 128), jnp.float32),
            jax.ShapeDtypeStruct((n, cout, 128), jnp.float32),
        ),
        grid=(n,),
        in_specs=[
            pl.BlockSpec((1, cin, p), lambda i: (i, 0, 0)),
            pl.BlockSpec((3, cout, 3 * cin), lambda i: (0, 0, 0)),
            pl.BlockSpec((2, cin, p), lambda i: (0, 0, 0)),
        ],
        out_specs=(
            pl.BlockSpec((1, cout, 128), lambda i: (i, 0, 0)),
            pl.BlockSpec((1, cout, 128), lambda i: (i, 0, 0)),
        ),
        compiler_params=pltpu.CompilerParams(
            dimension_semantics=("parallel",)),
    )(x3, w3, masks)

    sum_y = jnp.sum(stats1[:, :, 0], axis=0)
    sum_y2 = jnp.sum(stats2[:, :, 0], axis=0)
    mean = sum_y / m_total
    var = jnp.maximum(sum_y2 / m_total - mean * mean, 0.0)
    inv = jax.lax.rsqrt(var + _EPS)
    scale = gamma.astype(jnp.float32) * inv
    shift = beta.astype(jnp.float32) - mean * scale

    w3_scaled = w3 * scale[None, :, None]
    shift2d = jnp.broadcast_to(shift[:, None], (cout, 128))

    out3 = pl.pallas_call(
        functools.partial(_apply_kernel, hw=hw),
        out_shape=jax.ShapeDtypeStruct((n, cout, p), x.dtype),
        grid=(n,),
        in_specs=[
            pl.BlockSpec((1, cin, p), lambda i: (i, 0, 0)),
            pl.BlockSpec((3, cout, 3 * cin), lambda i: (0, 0, 0)),
            pl.BlockSpec((2, cin, p), lambda i: (0, 0, 0)),
            pl.BlockSpec((cout, 128), lambda i: (0, 0)),
        ],
        out_specs=pl.BlockSpec((1, cout, p), lambda i: (i, 0, 0)),
        compiler_params=pltpu.CompilerParams(
            dimension_semantics=("parallel",)),
    )(x3, w3_scaled, masks, shift2d)

    return out3.reshape(n, cout, h, w)


# speedup vs baseline: 4.7070x; 1.3909x over previous
import functools

import jax
import jax.numpy as jnp
from jax.experimental import pallas as pl
from jax.experimental.pallas import tpu as pltpu

_EPS = 1e-5


def _shift_lanes(x, k):
    return pltpu.roll(x, k % x.shape[1], 1)


def _conv_image(x, w_ref, m_ref, cat_ref, hw):
    cin = x.shape[0]
    cat_ref[0:cin] = _shift_lanes(x, 1) * m_ref[0]
    cat_ref[cin:2 * cin] = x
    cat_ref[2 * cin:3 * cin] = _shift_lanes(x, -1) * m_ref[1]
    xcat = cat_ref[...]

    a0 = jnp.dot(w_ref[1], xcat, preferred_element_type=jnp.float32)
    am = jnp.dot(w_ref[0], xcat, preferred_element_type=jnp.float32)
    ap = jnp.dot(w_ref[2], xcat, preferred_element_type=jnp.float32)

    h, w = hw
    p = h * w
    cout = a0.shape[0]
    z = jnp.zeros((cout, w), jnp.float32)
    return (a0
            + jnp.concatenate([z, am[:, :p - w]], axis=1)
            + jnp.concatenate([ap[:, w:], z], axis=1))


def _fused_kernel(x_ref, w_ref, m_ref, gb_ref, o_ref,
                  cat_ref, yc_ref, acc_ref, sc_ref, *, hw, n):
    t = pl.program_id(0)
    i = pl.program_id(1)
    h, w = hw
    p = h * w
    cout = o_ref.shape[1]

    @pl.when(t == 0)
    def _phase0():
        @pl.when(i == 0)
        def _init():
            acc_ref[...] = jnp.zeros_like(acc_ref)

        y = _conv_image(x_ref[0], w_ref, m_ref, cat_ref, hw)
        yc_ref[i] = y.astype(jnp.bfloat16)
        acc_ref[:, 0:1] += jnp.sum(y, axis=1, keepdims=True)
        acc_ref[:, 1:2] += jnp.sum(y * y, axis=1, keepdims=True)

    @pl.when(t == 1)
    def _phase1():
        @pl.when(i == 0)
        def _finalize():
            m_total = jnp.float32(n * p)
            mean = acc_ref[:, 0:1] / m_total
            var = jnp.maximum(acc_ref[:, 1:2] / m_total - mean * mean, 0.0)
            inv = jax.lax.rsqrt(var + _EPS)
            scale = gb_ref[0, :, 0:1] * inv
            sc_ref[:, 0:1] = scale
            sc_ref[:, 1:2] = gb_ref[1, :, 0:1] - mean * scale

        y = yc_ref[i].astype(jnp.float32)
        o_ref[0] = (y * jnp.broadcast_to(sc_ref[:, 0:1], (cout, p))
                    + jnp.broadcast_to(sc_ref[:, 1:2], (cout, p)))


def kernel(x, conv_w, gamma, beta):
    n, cin, h, w = x.shape
    cout = conv_w.shape[0]
    p = h * w

    x3 = x.reshape(n, cin, p)
    w3 = jnp.transpose(conv_w, (2, 0, 3, 1)).reshape(3, cout, 3 * cin)

    wcol = jnp.arange(p, dtype=jnp.int32) % w
    masks = jnp.stack([(wcol != 0), (wcol != w - 1)]).astype(jnp.float32)
    masks = jnp.broadcast_to(masks[:, None, :], (2, cin, p))

    gb = jnp.stack([
        jnp.broadcast_to(gamma.astype(jnp.float32)[:, None], (cout, 128)),
        jnp.broadcast_to(beta.astype(jnp.float32)[:, None], (cout, 128)),
    ])

    out3 = pl.pallas_call(
        functools.partial(_fused_kernel, hw=(h, w), n=n),
        out_shape=jax.ShapeDtypeStruct((n, cout, p), x.dtype),
        grid=(2, n),
        in_specs=[
            pl.BlockSpec((1, cin, p), lambda t, i: ((1 - t) * i, 0, 0)),
            pl.BlockSpec((3, cout, 3 * cin), lambda t, i: (0, 0, 0)),
            pl.BlockSpec((2, cin, p), lambda t, i: (0, 0, 0)),
            pl.BlockSpec((2, cout, 128), lambda t, i: (0, 0, 0)),
        ],
        out_specs=pl.BlockSpec((1, cout, p), lambda t, i: (t * i, 0, 0)),
        scratch_shapes=[
            pltpu.VMEM((3 * cin, p), jnp.float32),
            pltpu.VMEM((n, cout, p), jnp.bfloat16),
            pltpu.VMEM((cout, 128), jnp.float32),
            pltpu.VMEM((cout, 128), jnp.float32),
        ],
        compiler_params=pltpu.CompilerParams(
            dimension_semantics=("arbitrary", "arbitrary"),
            vmem_limit_bytes=52 * 1024 * 1024,
        ),
    )(x3, w3, masks, gb)

    return out3.reshape(n, cout, h, w)


# speedup vs baseline: 5.5701x; 1.1834x over previous
import functools

import jax
import jax.numpy as jnp
from jax.experimental import pallas as pl
from jax.experimental.pallas import tpu as pltpu

_EPS = 1e-5


def _shift_lanes(x, k):
    return pltpu.roll(x, k % x.shape[1], 1)


def _conv_image(x, w_ref, m_ref, cat_ref, hw):
    cin = x.shape[0]
    cat_ref[0:cin] = _shift_lanes(x, 1).astype(jnp.bfloat16) * m_ref[0]
    cat_ref[cin:2 * cin] = x.astype(jnp.bfloat16)
    cat_ref[2 * cin:3 * cin] = _shift_lanes(x, -1).astype(jnp.bfloat16) * m_ref[1]
    xcat = cat_ref[...]

    a0 = jnp.dot(w_ref[1], xcat, preferred_element_type=jnp.float32)
    am = jnp.dot(w_ref[0], xcat, preferred_element_type=jnp.float32)
    ap = jnp.dot(w_ref[2], xcat, preferred_element_type=jnp.float32)

    h, w = hw
    p = h * w
    cout = a0.shape[0]
    z = jnp.zeros((cout, w), jnp.float32)
    return (a0
            + jnp.concatenate([z, am[:, :p - w]], axis=1)
            + jnp.concatenate([ap[:, w:], z], axis=1))


def _fused_kernel(x_ref, w_ref, m_ref, gb_ref, o_ref,
                  cat_ref, yc_ref, acc_ref, sc_ref, *, hw, n, upi):
    t = pl.program_id(0)
    i = pl.program_id(1)
    h, w = hw
    p = h * w
    cout = o_ref.shape[2]

    @pl.when(t == 0)
    def _phase0():
        @pl.when(i == 0)
        def _init():
            acc_ref[...] = jnp.zeros_like(acc_ref)

        s1 = jnp.zeros((cout, 1), jnp.float32)
        s2 = jnp.zeros((cout, 1), jnp.float32)
        for u in range(upi):
            y = _conv_image(x_ref[0, u], w_ref, m_ref, cat_ref.at[u], hw)
            yc_ref[i * upi + u] = y.astype(jnp.bfloat16)
            s1 += jnp.sum(y, axis=1, keepdims=True)
            s2 += jnp.sum(y * y, axis=1, keepdims=True)
        acc_ref[:, 0:1] += s1
        acc_ref[:, 1:2] += s2

    @pl.when(t == 1)
    def _phase1():
        @pl.when(i == 0)
        def _finalize():
            m_total = jnp.float32(n * p)
            mean = acc_ref[:, 0:1] / m_total
            var = jnp.maximum(acc_ref[:, 1:2] / m_total - mean * mean, 0.0)
            inv = jax.lax.rsqrt(var + _EPS)
            scale = gb_ref[0, :, 0:1] * inv
            sc_ref[:, 0:1] = scale
            sc_ref[:, 1:2] = gb_ref[1, :, 0:1] - mean * scale

        for u in range(upi):
            y = yc_ref[i * upi + u].astype(jnp.float32)
            o_ref[0, u] = (y * jnp.broadcast_to(sc_ref[:, 0:1], (cout, p))
                           + jnp.broadcast_to(sc_ref[:, 1:2], (cout, p)))


def kernel(x, conv_w, gamma, beta):
    n, cin, h, w = x.shape
    cout = conv_w.shape[0]
    p = h * w
    upi = 2 if n % 2 == 0 else 1
    steps = n // upi

    x3 = x.reshape(steps, upi, cin, p)
    w3 = jnp.transpose(conv_w, (2, 0, 3, 1)).reshape(
        3, cout, 3 * cin).astype(jnp.bfloat16)

    wcol = jnp.arange(p, dtype=jnp.int32) % w
    masks = jnp.stack([(wcol != 0), (wcol != w - 1)]).astype(jnp.bfloat16)
    masks = jnp.broadcast_to(masks[:, None, :], (2, cin, p))

    gb = jnp.stack([
        jnp.broadcast_to(gamma.astype(jnp.float32)[:, None], (cout, 128)),
        jnp.broadcast_to(beta.astype(jnp.float32)[:, None], (cout, 128)),
    ])

    out3 = pl.pallas_call(
        functools.partial(_fused_kernel, hw=(h, w), n=n, upi=upi),
        out_shape=jax.ShapeDtypeStruct((steps, upi, cout, p), x.dtype),
        grid=(2, steps),
        in_specs=[
            pl.BlockSpec((1, upi, cin, p), lambda t, i: ((1 - t) * i, 0, 0, 0)),
            pl.BlockSpec((3, cout, 3 * cin), lambda t, i: (0, 0, 0)),
            pl.BlockSpec((2, cin, p), lambda t, i: (0, 0, 0)),
            pl.BlockSpec((2, cout, 128), lambda t, i: (0, 0, 0)),
        ],
        out_specs=pl.BlockSpec((1, upi, cout, p),
                               lambda t, i: (t * i, 0, 0, 0)),
        scratch_shapes=[
            pltpu.VMEM((upi, 3 * cin, p), jnp.bfloat16),
            pltpu.VMEM((n, cout, p), jnp.bfloat16),
            pltpu.VMEM((cout, 128), jnp.float32),
            pltpu.VMEM((cout, 128), jnp.float32),
        ],
        compiler_params=pltpu.CompilerParams(
            dimension_semantics=("arbitrary", "arbitrary"),
            vmem_limit_bytes=56 * 1024 * 1024,
            allow_input_fusion=[True, True, True, True],
        ),
    )(x3, w3, masks, gb)

    return out3.reshape(n, cout, h, w)
